# Initial kernel scaffold; baseline (speedup 1.0000x reference)
import jax
import jax.numpy as jnp
from jax.experimental import pallas as pl


def kernel(x, emb, W_ih_f, W_hh_f, b_ih_f, b_hh_f, W_ih_b, W_hh_b, b_ih_b, b_hh_b, W_ent, b_ent, W_len, b_len, W_cd, b_cd, W_pat, b_pat, W_g1, b_g1, W_g2, b_g2, W_g3, b_g3):
    raise NotImplementedError("write your pallas kernel here")


# speedup vs baseline: 2.6939x; 2.6939x over previous
import functools

import jax
import jax.numpy as jnp
from jax import lax
from jax.experimental import pallas as pl
from jax.experimental.pallas import tpu as pltpu
from jax.experimental.pallas import tpu_sc as plsc

B, L, V, D, H, NE = 4096, 50, 1000, 128, 32, 2
BL = B * L
BLK = 256
NC, NS = 2, 16
NW = NC * NS
PER_W = BL // NW
CH = 128
NCHUNK = PER_W // CH
NSPLIT = 2


TW = 384


def _gate_cols(gmat):
    return (jnp.concatenate([gmat[:, 0:H], gmat[:, H:2 * H],
                             gmat[:, 3 * H:4 * H]], axis=1),
            gmat[:, 2 * H:3 * H])


def _prep_body(emb_ref, wf_ref, bf_ref, wb_ref, bb_ref, cdt_ref,
               whhf_ref, whhb_ref, t_ref, w_ref):
    f32 = jnp.float32
    e = emb_ref[...]
    gf = jnp.dot(e, wf_ref[...], preferred_element_type=f32) + bf_ref[...]
    gb = jnp.dot(e, wb_ref[...], preferred_element_type=f32) + bb_ref[...]
    sf, tf_ = _gate_cols(gf)
    sb, tb_ = _gate_cols(gb)
    z = jnp.zeros((V, TW - 2 * 4 * H - 16), f32)
    t_ref[...] = jnp.concatenate([sf, sb, tf_, tb_, cdt_ref[...], z], axis=1)
    swf, twf = _gate_cols(whhf_ref[...])
    swb, twb = _gate_cols(whhb_ref[...])
    zh = jnp.zeros((H, 96), f32)
    zt = jnp.zeros((H, 32), f32)
    w_ref[...] = jnp.concatenate([
        jnp.concatenate([swf, zh, twf, zt], axis=1),
        jnp.concatenate([zh, swb, zt, twb], axis=1),
    ], axis=0)


def _prep_tables(emb, wf_t, bias_f, wb_t, bias_b, cd_t, whhf_t, whhb_t):
    return pl.pallas_call(
        _prep_body,
        out_shape=[
            jax.ShapeDtypeStruct((V, TW), jnp.float32),
            jax.ShapeDtypeStruct((2 * H, 8 * H), jnp.float32),
        ],
    )(emb, wf_t, bias_f, wb_t, bias_b, cd_t, whhf_t, whhb_t)


def _sc_gather_body(per_w, nchunk, t_hbm, idx_hbm, g_hbm,
                    idx0_v, idx1_v, rows0_v, rows1_v, sem0, sem1):
    wid = lax.axis_index("s") * NC + lax.axis_index("c")
    base = wid * per_w

    def pair(i, _):
        off0 = base + (2 * i) * CH
        off1 = base + (2 * i + 1) * CH
        pltpu.sync_copy(idx_hbm.at[pl.ds(off0, CH)], idx0_v)
        c0 = pltpu.async_copy(t_hbm.at[idx0_v], rows0_v, sem0)
        pltpu.sync_copy(idx_hbm.at[pl.ds(off1, CH)], idx1_v)
        c1 = pltpu.async_copy(t_hbm.at[idx1_v], rows1_v, sem1)
        c0.wait()
        pltpu.sync_copy(rows0_v, g_hbm.at[pl.ds(off0, CH)])
        c1.wait()
        pltpu.sync_copy(rows1_v, g_hbm.at[pl.ds(off1, CH)])
        return 0

    lax.fori_loop(0, nchunk // 2, pair, 0)


def _sc_gather(table, idx):
    rows = idx.shape[0]
    per_w = rows // NW
    fn = pl.kernel(
        functools.partial(_sc_gather_body, per_w, per_w // CH),
        mesh=plsc.VectorSubcoreMesh(core_axis_name="c", subcore_axis_name="s"),
        out_type=jax.ShapeDtypeStruct((rows, TW), jnp.float32),
        scratch_types=[
            pltpu.VMEM((CH,), jnp.int32),
            pltpu.VMEM((CH,), jnp.int32),
            pltpu.VMEM((CH, TW), jnp.float32),
            pltpu.VMEM((CH, TW), jnp.float32),
            pltpu.SemaphoreType.DMA,
            pltpu.SemaphoreType.DMA,
        ],
    )
    return fn(table, idx)


def _shift(a, k):
    blk = a.shape[0]
    return jnp.concatenate([a[:, k:], jnp.zeros((blk, k), a.dtype)], axis=1)


def _main_body(x_ref, g_ref, wcat_ref,
               went_ref, bent_ref, wlen_ref, blen_ref, bcd_ref,
               wpat_ref, bpat_ref, wg1_ref, bg1_ref, wg2_ref, bg2_ref,
               wg3_ref, bg3_ref, out_ref):
    f32 = jnp.float32
    x = x_ref[...]
    vf = (x != 0).astype(f32)
    nf = jnp.sum(vf, axis=1, keepdims=True)
    tot = jnp.maximum(nf, 1.0)

    m = jnp.zeros((BLK, L), f32)
    for s in range(L):
        m = m + (x == x[:, s:s + 1]).astype(f32) * vf[:, s:s + 1]
    se = jnp.sum(vf * jnp.log(m / tot + 1e-8), axis=1, keepdims=True)
    ent = jnp.where(nf <= 1.0, 0.0, -se / tot)

    xf = x.astype(f32)
    val = xf * vf
    v1 = _shift(val, 1)
    cnt = _shift(vf, 1)
    v2 = jnp.zeros_like(v1)
    for k in (1, 2, 4, 8, 16, 32):
        v1r = _shift(v1, k)
        v2r = _shift(v2, k)
        cr = _shift(cnt, k)
        nv1 = jnp.where(cnt >= 1.0, v1, v1r)
        nv2 = jnp.where(cnt >= 2.0, v2, jnp.where(cnt == 1.0, v1r, v2r))
        cnt = jnp.minimum(cnt + cr, 2.0)
        v1, v2 = nv1, nv2
    rep_c = jnp.sum(vf * (v1 == xf).astype(f32), axis=1, keepdims=True)
    inc_c = jnp.sum(vf * (v1 > xf).astype(f32), axis=1, keepdims=True)
    dec_c = jnp.sum(vf * ((v1 < xf) & (v1 > 0.0)).astype(f32), axis=1,
                    keepdims=True)
    per_c = jnp.sum(vf * (v2 == xf).astype(f32), axis=1, keepdims=True)
    d1 = jnp.maximum(nf - 1.0, 1.0)
    d2 = jnp.maximum(nf - 2.0, 1.0)
    gt1 = (nf >= 2.0).astype(f32)
    rep = gt1 * rep_c / d1
    inc = gt1 * inc_c / d1
    dec = gt1 * dec_c / d1
    per = (nf >= 4.0).astype(f32) * per_c / d2

    wcat = wcat_ref[...]
    h_cat = jnp.zeros((BLK, 2 * H), f32)
    c_cat = jnp.zeros((BLK, 2 * H), f32)
    cd_acc = jnp.zeros((BLK, 16), f32)
    for t in range(L):
        addend = jnp.concatenate([
            g_ref[:, t, 0:96],
            g_ref[:, L - 1 - t, 96:192],
            g_ref[:, t, 192:224],
            g_ref[:, L - 1 - t, 224:256],
        ], axis=1)
        pre = addend + jnp.dot(h_cat, wcat, preferred_element_type=f32)
        s = jax.nn.sigmoid(pre[:, 0:192])
        g_cat = jnp.tanh(pre[:, 192:256])
        i_cat = jnp.concatenate([s[:, 0:H], s[:, 3 * H:4 * H]], axis=1)
        f_cat = jnp.concatenate([s[:, H:2 * H], s[:, 4 * H:5 * H]], axis=1)
        o_cat = jnp.concatenate([s[:, 2 * H:3 * H], s[:, 5 * H:6 * H]], axis=1)
        c_cat = f_cat * c_cat + i_cat * g_cat
        h_cat = o_cat * jnp.tanh(c_cat)
        cd_acc = cd_acc + g_ref[:, t, 256:272] * vf[:, t:t + 1]
    h_f = h_cat[:, 0:H]
    h_b = h_cat[:, H:2 * H]

    ent_feat = (ent / 4.0) * went_ref[...] + bent_ref[...]
    len_feat = (nf / 40.0) * wlen_ref[...] + blen_ref[...]
    cd_feat = cd_acc / tot + bcd_ref[...]
    pat = jnp.concatenate([rep, inc, dec, per], axis=1)
    pat_feat = jnp.dot(pat, wpat_ref[...], preferred_element_type=f32) + bpat_ref[...]

    allf = jnp.concatenate([h_f, h_b, ent_feat, len_feat, cd_feat, pat_feat],
                           axis=1)
    h1 = jnp.maximum(
        jnp.dot(allf, wg1_ref[...], preferred_element_type=f32) + bg1_ref[...], 0.0)
    h2 = jnp.maximum(
        jnp.dot(h1, wg2_ref[...], preferred_element_type=f32) + bg2_ref[...], 0.0)
    z = jnp.dot(h2, wg3_ref[...], preferred_element_type=f32) + bg3_ref[...]
    z = z - jnp.max(z, axis=1, keepdims=True)
    e = jnp.exp(z)
    out_ref[...] = e / jnp.sum(e, axis=1, keepdims=True)


def _run_main(x, g, wcat, went, bent, wlen, blen, bcd,
              wpat_t, bpat, wg1_t, bg1, wg2_t, bg2, wg3_t, bg3,
              interpret=False):
    nb = x.shape[0] // BLK
    full = lambda shape: pl.BlockSpec(shape, lambda i: (0,) * len(shape))
    return pl.pallas_call(
        _main_body,
        grid=(nb,),
        in_specs=[
            pl.BlockSpec((BLK, L), lambda i: (i, 0)),
            pl.BlockSpec((BLK, L, TW), lambda i: (i, 0, 0)),
            full((2 * H, 8 * H)),
            full((1, 8)), full((1, 8)),
            full((1, 8)), full((1, 8)),
            full((1, 16)),
            full((4, 8)), full((1, 8)),
            full((104, 64)), full((1, 64)),
            full((64, 32)), full((1, 32)),
            full((32, NE)), full((1, NE)),
        ],
        out_specs=pl.BlockSpec((BLK, NE), lambda i: (i, 0)),
        out_shape=jax.ShapeDtypeStruct((x.shape[0], NE), jnp.float32),
        interpret=interpret,
    )(x, g, wcat, went, bent, wlen, blen, bcd,
      wpat_t, bpat, wg1_t, bg1, wg2_t, bg2, wg3_t, bg3)


def kernel(x, emb, W_ih_f, W_hh_f, b_ih_f, b_hh_f, W_ih_b, W_hh_b, b_ih_b,
           b_hh_b, W_ent, b_ent, W_len, b_len, W_cd, b_cd, W_pat, b_pat,
           W_g1, b_g1, W_g2, b_g2, W_g3, b_g3):
    x = x.astype(jnp.int32)
    table, wcat = _prep_tables(
        emb,
        W_ih_f.T,
        (b_ih_f + b_hh_f).reshape(1, 4 * H),
        W_ih_b.T,
        (b_ih_b + b_hh_b).reshape(1, 4 * H),
        W_cd.T,
        W_hh_f.T, W_hh_b.T,
    )
    bh = B // NSPLIT
    outs = []
    for i in range(NSPLIT):
        xh = x[i * bh:(i + 1) * bh]
        g = _sc_gather(table, xh.reshape(bh * L))
        outs.append(_run_main(
            xh,
            g.reshape(bh, L, TW),
            wcat,
            W_ent.T, b_ent.reshape(1, 8),
            W_len.T, b_len.reshape(1, 8),
            b_cd.reshape(1, 16),
            W_pat.T, b_pat.reshape(1, 8),
            W_g1.T, b_g1.reshape(1, 64),
            W_g2.T, b_g2.reshape(1, 32),
            W_g3.T, b_g3.reshape(1, NE),
        ))
    return jnp.concatenate(outs, axis=0) if NSPLIT > 1 else outs[0]
